# Initial kernel scaffold; baseline (speedup 1.0000x reference)
import jax
import jax.numpy as jnp
from jax.experimental import pallas as pl


def kernel(x, central_species, neighbor_species, neighbors_index, nums, mask, batch_ids, species_emb, W_r, W1, w_out):
    raise NotImplementedError("write your pallas kernel here")


# speedup vs baseline: 6.5665x; 6.5665x over previous
import functools

import jax
import jax.numpy as jnp
from jax import lax
from jax.experimental import pallas as pl
from jax.experimental.pallas import tpu as pltpu
from jax.experimental.pallas import tpu_sc as plsc

N = 16384
M = 32
D = 128
S = 16
B = 16

NC, NS, L = 2, 16, 16
NW = NC * NS
APW = N // NW
A = 8
SUP = 32
DC = D // L


def _h_body(cs_ref, semb_ref, h_ref):
    cs = cs_ref[...]
    oh = (cs == lax.broadcasted_iota(jnp.int32, (cs.shape[0], S), 1))
    h_ref[...] = jnp.dot(oh.astype(jnp.float32), semb_ref[...],
                         preferred_element_type=jnp.float32)


def _build_h(central_species, species_emb):
    BLK = 2048
    return pl.pallas_call(
        _h_body,
        grid=(N // BLK,),
        in_specs=[pl.BlockSpec((BLK, 1), lambda i: (i, 0)),
                  pl.BlockSpec((S, D), lambda i: (0, 0))],
        out_specs=pl.BlockSpec((BLK, D), lambda i: (i, 0)),
        out_shape=jax.ShapeDtypeStruct((N, D), jnp.float32),
    )(central_species.reshape(N, 1), species_emb)


_MESH = plsc.VectorSubcoreMesh(core_axis_name="c", subcore_axis_name="s",
                               num_cores=NC, num_subcores=NS)


@functools.partial(
    pl.kernel,
    out_type=jax.ShapeDtypeStruct((N, D), jnp.float32),
    mesh=_MESH,
    scratch_types=[
        pltpu.VMEM((SUP // 4, 128), jnp.int32),
        pltpu.VMEM((A * M, D), jnp.float32),
        pltpu.VMEM((SUP, 128), jnp.float32),
        pltpu.VMEM((SUP, M), jnp.int32),
        pltpu.VMEM((SUP, M), jnp.float32),
        pltpu.VMEM((S, D), jnp.float32),
        pltpu.VMEM((3, D), jnp.float32),
        pltpu.VMEM((SUP, D), jnp.float32),
        pltpu.SemaphoreType.DMA,
    ],
)
def _sc_pool(h_hbm, idx2_hbm, xp_hbm, ns_hbm, vs_hbm, semb_hbm,
             wr_hbm, out_hbm,
             idx_v, rows_v, xp_v, ns_v, vs_v, semb_v, wr_v, out_v,
             sem):
    wid = lax.axis_index("s") * NC + lax.axis_index("c")
    base0 = wid * APW

    pltpu.sync_copy(semb_hbm, semb_v)
    pltpu.sync_copy(wr_hbm, wr_v)
    wr = [[wr_v[c, pl.ds(dc * L, L)] for dc in range(DC)] for c in range(3)]

    def sup_body(si, _):
        base = pl.multiple_of(base0 + si * SUP, SUP)
        pltpu.sync_copy(idx2_hbm.at[pl.ds(pl.multiple_of(base // 4, 8), SUP // 4)], idx_v)
        pltpu.sync_copy(xp_hbm.at[pl.ds(base, SUP)], xp_v)
        pltpu.sync_copy(ns_hbm.at[pl.ds(base, SUP)], ns_v)
        pltpu.sync_copy(vs_hbm.at[pl.ds(base, SUP)], vs_v)

        for j in range(SUP // A):
            cp0 = pltpu.async_copy(h_hbm.at[idx_v.at[2 * j]],
                                   rows_v.at[pl.ds(0, 128)], sem)
            cp1 = pltpu.async_copy(h_hbm.at[idx_v.at[2 * j + 1]],
                                   rows_v.at[pl.ds(128, 128)], sem)
            cp0.wait()
            cp1.wait()

            def atom_body(a, _):
                aw = a + j * A
                xc = [xp_v[aw, pl.ds(k * L, L)] for k in range(6)]
                vc = [vs_v[aw, pl.ds(0, L)], vs_v[aw, pl.ds(L, L)]]
                nsc = [ns_v[aw, pl.ds(0, L)], ns_v[aw, pl.ds(L, L)]]
                acc = [jnp.zeros((L,), jnp.float32) for _ in range(DC)]
                for m in range(M):
                    g, lane = m // L, m % L
                    x0 = jnp.full((L,), xc[g][lane], jnp.float32)
                    x1 = jnp.full((L,), xc[2 + g][lane], jnp.float32)
                    x2 = jnp.full((L,), xc[4 + g][lane], jnp.float32)
                    vv = jnp.full((L,), vc[g][lane], jnp.float32)
                    s = nsc[g][lane]
                    r = a * M + m
                    for dc in range(DC):
                        t = rows_v[r, pl.ds(dc * L, L)] + semb_v[s, pl.ds(dc * L, L)]
                        t = t + x0 * wr[0][dc]
                        t = t + x1 * wr[1][dc]
                        t = t + x2 * wr[2][dc]
                        t = jnp.maximum(t, 0.0)
                        acc[dc] = acc[dc] + vv * t
                for dc in range(DC):
                    out_v[aw, pl.ds(dc * L, L)] = acc[dc]
                return 0

            lax.fori_loop(0, A, atom_body, 0)

        pltpu.sync_copy(out_v, out_hbm.at[pl.ds(base, SUP)])
        return 0

    lax.fori_loop(0, APW // SUP, sup_body, 0)


def _tail_body(h_ref, p_ref, bid_ref, w1_ref, wo_ref, out_ref):
    i = pl.program_id(0)
    z = jnp.dot(h_ref[...] + p_ref[...], w1_ref[...],
                preferred_element_type=jnp.float32)
    z = jnp.maximum(z, 0.0)
    atom_e = jnp.dot(z, wo_ref[...], preferred_element_type=jnp.float32)
    oh = (bid_ref[...] == lax.broadcasted_iota(jnp.int32, (bid_ref.shape[0], B), 1))
    contrib = jnp.dot(atom_e.reshape(1, -1), oh.astype(jnp.float32),
                      preferred_element_type=jnp.float32)

    @pl.when(i == 0)
    def _():
        out_ref[...] = jnp.zeros_like(out_ref)

    out_ref[...] += contrib


def _tail(h, pooled, batch_ids, W1, w_out):
    BLK = 2048
    out = pl.pallas_call(
        _tail_body,
        grid=(N // BLK,),
        in_specs=[pl.BlockSpec((BLK, D), lambda i: (i, 0)),
                  pl.BlockSpec((BLK, D), lambda i: (i, 0)),
                  pl.BlockSpec((BLK, 1), lambda i: (i, 0)),
                  pl.BlockSpec((D, D), lambda i: (0, 0)),
                  pl.BlockSpec((D, 1), lambda i: (0, 0))],
        out_specs=pl.BlockSpec((1, B), lambda i: (0, 0)),
        out_shape=jax.ShapeDtypeStruct((1, B), jnp.float32),
    )(h, pooled, batch_ids.reshape(N, 1), W1, w_out.reshape(D, 1))
    return out.reshape(B)


def kernel(x, central_species, neighbor_species, neighbors_index, nums, mask,
           batch_ids, species_emb, W_r, W1, w_out):
    cs = central_species.astype(jnp.int32)
    ns = neighbor_species.astype(jnp.int32)
    idx2 = neighbors_index.astype(jnp.int32).reshape(N // 4, 128)
    bid = batch_ids.astype(jnp.int32)
    inv_denom = 1.0 / jnp.maximum(nums, 1).astype(jnp.float32)
    vs = (~mask).astype(jnp.float32) * inv_denom[:, None]
    xp = jnp.concatenate(
        [x[:, :, 0], x[:, :, 1], x[:, :, 2],
         jnp.zeros((N, M), jnp.float32)], axis=1)

    h = _build_h(cs, species_emb)
    pooled = _sc_pool(h, idx2, xp, ns, vs, species_emb, W_r)
    return _tail(h, pooled, bid, W1, w_out)


# speedup vs baseline: 9.2320x; 1.4059x over previous
import functools

import jax
import jax.numpy as jnp
from jax import lax
from jax.experimental import pallas as pl
from jax.experimental.pallas import tpu as pltpu
from jax.experimental.pallas import tpu_sc as plsc

N = 16384
M = 32
D = 128
S = 16
B = 16

NC, NS, L = 2, 16, 16
NW = NC * NS
APW = N // NW
A = 8
SUP = 32
DC = D // L


def _h_body(cs_ref, semb_ref, h_ref):
    cs = cs_ref[...]
    oh = (cs == lax.broadcasted_iota(jnp.int32, (cs.shape[0], S), 1))
    h_ref[...] = jnp.dot(oh.astype(jnp.float32), semb_ref[...],
                         preferred_element_type=jnp.float32)


def _build_h(central_species, species_emb):
    BLK = 2048
    return pl.pallas_call(
        _h_body,
        grid=(N // BLK,),
        in_specs=[pl.BlockSpec((BLK, 1), lambda i: (i, 0)),
                  pl.BlockSpec((S, D), lambda i: (0, 0))],
        out_specs=pl.BlockSpec((BLK, D), lambda i: (i, 0)),
        out_shape=jax.ShapeDtypeStruct((N, D), jnp.float32),
    )(central_species.reshape(N, 1), species_emb)


_MESH = plsc.VectorSubcoreMesh(core_axis_name="c", subcore_axis_name="s",
                               num_cores=NC, num_subcores=NS)


@functools.partial(
    pl.kernel,
    out_type=jax.ShapeDtypeStruct((N, D), jnp.float32),
    mesh=_MESH,
    scratch_types=[
        pltpu.VMEM((2, SUP // 4, 128), jnp.int32),
        pltpu.VMEM((2, A * M, D), jnp.float32),
        pltpu.VMEM((2, SUP, 128), jnp.float32),
        pltpu.VMEM((2, SUP, M), jnp.int32),
        pltpu.VMEM((2, SUP, M), jnp.float32),
        pltpu.VMEM((S, D), jnp.float32),
        pltpu.VMEM((3, D), jnp.float32),
        pltpu.VMEM((SUP, D), jnp.float32),
        pltpu.SemaphoreType.DMA,
        pltpu.SemaphoreType.DMA,
        pltpu.SemaphoreType.DMA,
    ],
)
def _sc_pool(h_hbm, idx2_hbm, xp_hbm, ns_hbm, vs_hbm, semb_hbm,
             wr_hbm, out_hbm,
             idx_v, rows_v, xp_v, ns_v, vs_v, semb_v, wr_v, out_v,
             sem0, sem1, sem_in):
    wid = lax.axis_index("s") * NC + lax.axis_index("c")
    base0 = wid * APW
    NSUP = APW // SUP
    rsem = [sem0, sem1]

    pltpu.sync_copy(semb_hbm, semb_v)
    pltpu.sync_copy(wr_hbm, wr_v)
    wr = [[wr_v[c, pl.ds(dc * L, L)] for dc in range(DC)] for c in range(3)]

    def in_copies(sup_i, buf):
        base = pl.multiple_of(sup_i * SUP, SUP)
        return [
            pltpu.make_async_copy(
                idx2_hbm.at[pl.ds(pl.multiple_of(base0 // 4 + base // 4, 8), SUP // 4)],
                idx_v.at[buf], sem_in),
            pltpu.make_async_copy(xp_hbm.at[pl.ds(base0 + base, SUP)],
                                  xp_v.at[buf], sem_in),
            pltpu.make_async_copy(ns_hbm.at[pl.ds(base0 + base, SUP)],
                                  ns_v.at[buf], sem_in),
            pltpu.make_async_copy(vs_hbm.at[pl.ds(base0 + base, SUP)],
                                  vs_v.at[buf], sem_in),
        ]

    def gather(sup_buf, j, rbuf):
        pltpu.async_copy(h_hbm.at[idx_v.at[sup_buf, 2 * j]],
                         rows_v.at[rbuf, pl.ds(0, 128)], rsem[rbuf])
        pltpu.async_copy(h_hbm.at[idx_v.at[sup_buf, 2 * j + 1]],
                         rows_v.at[rbuf, pl.ds(128, 128)], rsem[rbuf])

    def wait_rows(j, rbuf):
        pltpu.make_async_copy(h_hbm.at[idx_v.at[0, 0]],
                              rows_v.at[rbuf, pl.ds(0, 128)], rsem[rbuf]).wait()
        pltpu.make_async_copy(h_hbm.at[idx_v.at[0, 0]],
                              rows_v.at[rbuf, pl.ds(128, 128)], rsem[rbuf]).wait()

    for cp in in_copies(0, 0):
        cp.start()
        cp.wait()
    gather(0, 0, 0)

    def sup_body(si, _):
        pi = lax.rem(si, 2)
        qi = 1 - pi

        @pl.when(si + 1 < NSUP)
        def _():
            for cp in in_copies(si + 1, qi):
                cp.start()

        for j in range(SUP // A):
            rb = j % 2
            wait_rows(j, rb)
            if j < SUP // A - 1:
                gather(pi, j + 1, 1 - rb)
            else:
                @pl.when(si + 1 < NSUP)
                def _():
                    for cp in in_copies(si + 1, qi):
                        cp.wait()
                    gather(qi, 0, 1 - rb)

            def atom_body(a, _):
                aw = a + j * A
                xc = [xp_v[pi, aw, pl.ds(k * L, L)] for k in range(6)]
                vc = [vs_v[pi, aw, pl.ds(0, L)], vs_v[pi, aw, pl.ds(L, L)]]
                nsc = [ns_v[pi, aw, pl.ds(0, L)], ns_v[pi, aw, pl.ds(L, L)]]
                acc = [jnp.zeros((L,), jnp.float32) for _ in range(DC)]
                for m in range(M):
                    g, lane = m // L, m % L
                    x0 = jnp.full((L,), xc[g][lane], jnp.float32)
                    x1 = jnp.full((L,), xc[2 + g][lane], jnp.float32)
                    x2 = jnp.full((L,), xc[4 + g][lane], jnp.float32)
                    vv = jnp.full((L,), vc[g][lane], jnp.float32)
                    s = nsc[g][lane]
                    r = a * M + m
                    for dc in range(DC):
                        t = rows_v[rb, r, pl.ds(dc * L, L)] + semb_v[s, pl.ds(dc * L, L)]
                        t = t + x0 * wr[0][dc]
                        t = t + x1 * wr[1][dc]
                        t = t + x2 * wr[2][dc]
                        t = jnp.maximum(t, 0.0)
                        acc[dc] = acc[dc] + vv * t
                for dc in range(DC):
                    out_v[aw, pl.ds(dc * L, L)] = acc[dc]
                return 0

            lax.fori_loop(0, A, atom_body, 0)

        base = pl.multiple_of(base0 + si * SUP, SUP)
        pltpu.sync_copy(out_v, out_hbm.at[pl.ds(base, SUP)])
        return 0

    lax.fori_loop(0, NSUP, sup_body, 0)


def _tail_body(h_ref, p_ref, bid_ref, w1_ref, wo_ref, out_ref):
    i = pl.program_id(0)
    z = jnp.dot(h_ref[...] + p_ref[...], w1_ref[...],
                preferred_element_type=jnp.float32)
    z = jnp.maximum(z, 0.0)
    atom_e = jnp.dot(z, wo_ref[...], preferred_element_type=jnp.float32)
    oh = (bid_ref[...] == lax.broadcasted_iota(jnp.int32, (bid_ref.shape[0], B), 1))
    contrib = jnp.dot(atom_e.reshape(1, -1), oh.astype(jnp.float32),
                      preferred_element_type=jnp.float32)

    @pl.when(i == 0)
    def _():
        out_ref[...] = jnp.zeros_like(out_ref)

    out_ref[...] += contrib


def _tail(h, pooled, batch_ids, W1, w_out):
    BLK = 2048
    out = pl.pallas_call(
        _tail_body,
        grid=(N // BLK,),
        in_specs=[pl.BlockSpec((BLK, D), lambda i: (i, 0)),
                  pl.BlockSpec((BLK, D), lambda i: (i, 0)),
                  pl.BlockSpec((BLK, 1), lambda i: (i, 0)),
                  pl.BlockSpec((D, D), lambda i: (0, 0)),
                  pl.BlockSpec((D, 1), lambda i: (0, 0))],
        out_specs=pl.BlockSpec((1, B), lambda i: (0, 0)),
        out_shape=jax.ShapeDtypeStruct((1, B), jnp.float32),
    )(h, pooled, batch_ids.reshape(N, 1), W1, w_out.reshape(D, 1))
    return out.reshape(B)


def kernel(x, central_species, neighbor_species, neighbors_index, nums, mask,
           batch_ids, species_emb, W_r, W1, w_out):
    cs = central_species.astype(jnp.int32)
    ns = neighbor_species.astype(jnp.int32)
    idx2 = neighbors_index.astype(jnp.int32).reshape(N // 4, 128)
    bid = batch_ids.astype(jnp.int32)
    inv_denom = 1.0 / jnp.maximum(nums, 1).astype(jnp.float32)
    vs = (~mask).astype(jnp.float32) * inv_denom[:, None]
    xp = jnp.concatenate(
        [x[:, :, 0], x[:, :, 1], x[:, :, 2],
         jnp.zeros((N, M), jnp.float32)], axis=1)

    h = _build_h(cs, species_emb)
    pooled = _sc_pool(h, idx2, xp, ns, vs, species_emb, W_r)
    return _tail(h, pooled, bid, W1, w_out)


# speedup vs baseline: 9.5338x; 1.0327x over previous
import functools

import jax
import jax.numpy as jnp
from jax import lax
from jax.experimental import pallas as pl
from jax.experimental.pallas import tpu as pltpu
from jax.experimental.pallas import tpu_sc as plsc

N = 16384
M = 32
D = 128
S = 16
B = 16

NC, NS, L = 2, 16, 16
NW = NC * NS
APW = N // NW
A = 8
SUP = 32
DC = D // L


def _h_body(cs_ref, semb_ref, h_ref):
    cs = cs_ref[...]
    oh = (cs == lax.broadcasted_iota(jnp.int32, (cs.shape[0], S), 1))
    h_ref[...] = jnp.dot(oh.astype(jnp.float32), semb_ref[...],
                         preferred_element_type=jnp.float32)


def _build_h(central_species, species_emb):
    BLK = 2048
    return pl.pallas_call(
        _h_body,
        grid=(N // BLK,),
        in_specs=[pl.BlockSpec((BLK, 1), lambda i: (i, 0)),
                  pl.BlockSpec((S, D), lambda i: (0, 0))],
        out_specs=pl.BlockSpec((BLK, D), lambda i: (i, 0)),
        out_shape=jax.ShapeDtypeStruct((N, D), jnp.float32),
    )(central_species.reshape(N, 1), species_emb)


_MESH = plsc.VectorSubcoreMesh(core_axis_name="c", subcore_axis_name="s",
                               num_cores=NC, num_subcores=NS)


@functools.partial(
    pl.kernel,
    out_type=jax.ShapeDtypeStruct((N, D), jnp.float32),
    mesh=_MESH,
    scratch_types=[
        pltpu.VMEM((2, SUP // 4, 128), jnp.int32),
        pltpu.VMEM((2, A * M, D), jnp.float32),
        pltpu.VMEM((2, SUP, 128), jnp.float32),
        pltpu.VMEM((2, SUP, M), jnp.int32),
        pltpu.VMEM((S + 1, D), jnp.float32),
        pltpu.VMEM((3, D), jnp.float32),
        pltpu.VMEM((SUP, D), jnp.float32),
        pltpu.SemaphoreType.DMA,
        pltpu.SemaphoreType.DMA,
        pltpu.SemaphoreType.DMA,
    ],
)
def _sc_pool(h_hbm, idx2_hbm, xp_hbm, ns_hbm, semb_hbm,
             wr_hbm, out_hbm,
             idx_v, rows_v, xp_v, ns_v, semb_v, wr_v, out_v,
             sem0, sem1, sem_in):
    wid = lax.axis_index("s") * NC + lax.axis_index("c")
    base0 = wid * APW
    NSUP = APW // SUP
    rsem = [sem0, sem1]

    pltpu.sync_copy(semb_hbm, semb_v)
    pltpu.sync_copy(wr_hbm, wr_v)
    wr = [[wr_v[c, pl.ds(dc * L, L)] for dc in range(DC)] for c in range(3)]

    def in_copies(sup_i, buf):
        base = pl.multiple_of(sup_i * SUP, SUP)
        return [
            pltpu.make_async_copy(
                idx2_hbm.at[pl.ds(pl.multiple_of(base0 // 4 + base // 4, 8), SUP // 4)],
                idx_v.at[buf], sem_in),
            pltpu.make_async_copy(xp_hbm.at[pl.ds(base0 + base, SUP)],
                                  xp_v.at[buf], sem_in),
            pltpu.make_async_copy(ns_hbm.at[pl.ds(base0 + base, SUP)],
                                  ns_v.at[buf], sem_in),
        ]

    def gather(sup_buf, j, rbuf):
        pltpu.async_copy(h_hbm.at[idx_v.at[sup_buf, 2 * j]],
                         rows_v.at[rbuf, pl.ds(0, 128)], rsem[rbuf])
        pltpu.async_copy(h_hbm.at[idx_v.at[sup_buf, 2 * j + 1]],
                         rows_v.at[rbuf, pl.ds(128, 128)], rsem[rbuf])

    def wait_rows(j, rbuf):
        pltpu.make_async_copy(h_hbm.at[idx_v.at[0, 0]],
                              rows_v.at[rbuf, pl.ds(0, 128)], rsem[rbuf]).wait()
        pltpu.make_async_copy(h_hbm.at[idx_v.at[0, 0]],
                              rows_v.at[rbuf, pl.ds(128, 128)], rsem[rbuf]).wait()

    for cp in in_copies(0, 0):
        cp.start()
        cp.wait()
    gather(0, 0, 0)

    def sup_body(si, _):
        pi = lax.rem(si, 2)
        qi = 1 - pi

        @pl.when(si + 1 < NSUP)
        def _():
            for cp in in_copies(si + 1, qi):
                cp.start()

        for j in range(SUP // A):
            rb = j % 2
            wait_rows(j, rb)
            if j < SUP // A - 1:
                gather(pi, j + 1, 1 - rb)
            else:
                @pl.when(si + 1 < NSUP)
                def _():
                    for cp in in_copies(si + 1, qi):
                        cp.wait()
                    gather(qi, 0, 1 - rb)

            def atom_body(a, _):
                aw = a + j * A
                xc = [xp_v[pi, aw, pl.ds(k * L, L)] for k in range(7)]
                nsc = [ns_v[pi, aw, pl.ds(0, L)], ns_v[pi, aw, pl.ds(L, L)]]
                iv = jnp.full((L,), xc[6][0], jnp.float32)
                acc = [jnp.zeros((L,), jnp.float32) for _ in range(DC)]
                for m in range(M):
                    g, lane = m // L, m % L
                    x0 = jnp.full((L,), xc[g][lane], jnp.float32)
                    x1 = jnp.full((L,), xc[2 + g][lane], jnp.float32)
                    x2 = jnp.full((L,), xc[4 + g][lane], jnp.float32)
                    s = nsc[g][lane]
                    r = a * M + m
                    for dc in range(DC):
                        t = rows_v[rb, r, pl.ds(dc * L, L)] + semb_v[s, pl.ds(dc * L, L)]
                        t = t + x0 * wr[0][dc]
                        t = t + x1 * wr[1][dc]
                        t = t + x2 * wr[2][dc]
                        t = jnp.maximum(t, 0.0)
                        acc[dc] = acc[dc] + t
                for dc in range(DC):
                    out_v[aw, pl.ds(dc * L, L)] = acc[dc] * iv
                return 0

            lax.fori_loop(0, A, atom_body, 0)

        base = pl.multiple_of(base0 + si * SUP, SUP)
        pltpu.sync_copy(out_v, out_hbm.at[pl.ds(base, SUP)])
        return 0

    lax.fori_loop(0, NSUP, sup_body, 0)


def _tail_body(h_ref, p_ref, bid_ref, w1_ref, wo_ref, out_ref):
    i = pl.program_id(0)
    z = jnp.dot(h_ref[...] + p_ref[...], w1_ref[...],
                preferred_element_type=jnp.float32)
    z = jnp.maximum(z, 0.0)
    atom_e = jnp.dot(z, wo_ref[...], preferred_element_type=jnp.float32)
    oh = (bid_ref[...] == lax.broadcasted_iota(jnp.int32, (bid_ref.shape[0], B), 1))
    contrib = jnp.dot(atom_e.reshape(1, -1), oh.astype(jnp.float32),
                      preferred_element_type=jnp.float32)

    @pl.when(i == 0)
    def _():
        out_ref[...] = jnp.zeros_like(out_ref)

    out_ref[...] += contrib


def _tail(h, pooled, batch_ids, W1, w_out):
    BLK = 2048
    out = pl.pallas_call(
        _tail_body,
        grid=(N // BLK,),
        in_specs=[pl.BlockSpec((BLK, D), lambda i: (i, 0)),
                  pl.BlockSpec((BLK, D), lambda i: (i, 0)),
                  pl.BlockSpec((BLK, 1), lambda i: (i, 0)),
                  pl.BlockSpec((D, D), lambda i: (0, 0)),
                  pl.BlockSpec((D, 1), lambda i: (0, 0))],
        out_specs=pl.BlockSpec((1, B), lambda i: (0, 0)),
        out_shape=jax.ShapeDtypeStruct((1, B), jnp.float32),
    )(h, pooled, batch_ids.reshape(N, 1), W1, w_out.reshape(D, 1))
    return out.reshape(B)


def kernel(x, central_species, neighbor_species, neighbors_index, nums, mask,
           batch_ids, species_emb, W_r, W1, w_out):
    cs = central_species.astype(jnp.int32)
    ns = neighbor_species.astype(jnp.int32)
    idx2 = neighbors_index.astype(jnp.int32).reshape(N // 4, 128)
    bid = batch_ids.astype(jnp.int32)
    ns_p = jnp.where(mask, S, ns)
    semb_p = jnp.concatenate(
        [species_emb, jnp.full((1, D), -1e30, jnp.float32)], axis=0)
    inv_denom = 1.0 / jnp.maximum(nums, 1).astype(jnp.float32)
    xp = jnp.concatenate(
        [x[:, :, 0], x[:, :, 1], x[:, :, 2], inv_denom[:, None],
         jnp.zeros((N, M - 1), jnp.float32)], axis=1)

    h = _build_h(cs, species_emb)
    pooled = _sc_pool(h, idx2, xp, ns_p, semb_p, W_r)
    return _tail(h, pooled, bid, W1, w_out)
